# baseline (speedup 1.0000x reference)
import functools

import jax
import jax.numpy as jnp
from jax import lax
from jax.experimental import pallas as pl
from jax.experimental.pallas import tpu as pltpu
from jax.experimental.pallas import tpu_sc as plsc

N = 10000
E = 320000
D = 128
D_FF = 256
D_OUT = 112
EPS = 1e-7

NC = 2
NS = 16
CHUNK = 128
CHUNKS = 160
IGRP = 8
EPW = CHUNK * CHUNKS
EPAD = NS * EPW
NPAD = 10112
ROWS_PER_SUB = NPAD // NS
OUT_ROWS_PER_SUB = 624
BN = 1000
GRID = N // BN


def _ln_relu(h, w, b):
    mu = jnp.mean(h, axis=-1, keepdims=True)
    var = jnp.mean((h - mu) ** 2, axis=-1, keepdims=True)
    return jax.nn.relu((h - mu) * lax.rsqrt(var + 1e-5) * w + b)


def _tables(t):
    m = t + EPS
    em = jnp.exp(m)
    return em, em * m


def _enc_pre_body(x_ref, w_ref, b_ref, lw_ref, lb_ref,
                  h_ref, t_ref, em_ref, p_ref):
    h = jnp.dot(x_ref[...], w_ref[...], preferred_element_type=jnp.float32)
    h = h + b_ref[...]
    h_ref[...] = h
    t = _ln_relu(h, lw_ref[...], lb_ref[...])
    t_ref[...] = t
    em, p = _tables(t)
    em_ref[...] = em
    p_ref[...] = p


def _post_body(h_ref, t_ref, den_ref, num_ref, w1_ref, b1_ref, w2_ref, b2_ref):
    agg = num_ref[...] / (den_ref[...] + 1e-16)
    out = agg + t_ref[...]
    h1 = jax.nn.relu(
        jnp.dot(out, w1_ref[...], preferred_element_type=jnp.float32)
        + b1_ref[...])
    conv = jnp.dot(h1, w2_ref[...], preferred_element_type=jnp.float32)
    return h_ref[...] + conv + b2_ref[...]


def _post_pre_body(h_ref, t_ref, den_ref, num_ref, w1_ref, b1_ref, w2_ref,
                   b2_ref, lw_ref, lb_ref, hn_ref, tn_ref, em_ref, p_ref):
    hn = _post_body(h_ref, t_ref, den_ref, num_ref, w1_ref, b1_ref, w2_ref,
                    b2_ref)
    hn_ref[...] = hn
    tn = _ln_relu(hn, lw_ref[...], lb_ref[...])
    tn_ref[...] = tn
    em, p = _tables(tn)
    em_ref[...] = em
    p_ref[...] = p


def _post_cls_body(h_ref, t_ref, den_ref, num_ref, w1_ref, b1_ref, w2_ref,
                   b2_ref, wc_ref, bc_ref, o_ref):
    hn = _post_body(h_ref, t_ref, den_ref, num_ref, w1_ref, b1_ref, w2_ref,
                    b2_ref)
    o_ref[...] = (jnp.dot(hn, wc_ref[...], preferred_element_type=jnp.float32)
                  + bc_ref[...])


def _row_spec(d=D):
    return pl.BlockSpec((BN, d), lambda i: (i, 0))


def _full_spec(shape):
    return pl.BlockSpec(shape, lambda i: tuple(0 for _ in shape))


def _enc_pre(x, w, b, lw, lb):
    f32 = jnp.float32
    return pl.pallas_call(
        _enc_pre_body,
        grid=(GRID,),
        in_specs=[_row_spec(), _full_spec((D, D)), _full_spec((1, D)),
                  _full_spec((1, D)), _full_spec((1, D))],
        out_specs=[_row_spec(), _row_spec(), _row_spec(), _row_spec()],
        out_shape=[jax.ShapeDtypeStruct((N, D), f32)] * 4,
    )(x, w, b, lw, lb)


def _post_pre(h, t, den, num, w1, b1, w2, b2, lw, lb):
    f32 = jnp.float32
    return pl.pallas_call(
        _post_pre_body,
        grid=(GRID,),
        in_specs=[_row_spec(), _row_spec(), _row_spec(), _row_spec(),
                  _full_spec((D, D_FF)), _full_spec((1, D_FF)),
                  _full_spec((D_FF, D)), _full_spec((1, D)),
                  _full_spec((1, D)), _full_spec((1, D))],
        out_specs=[_row_spec(), _row_spec(), _row_spec(), _row_spec()],
        out_shape=[jax.ShapeDtypeStruct((N, D), f32)] * 4,
    )(h, t, den, num, w1, b1, w2, b2, lw, lb)


def _post_cls(h, t, den, num, w1, b1, w2, b2, wc, bc):
    return pl.pallas_call(
        _post_cls_body,
        grid=(GRID,),
        in_specs=[_row_spec(), _row_spec(), _row_spec(), _row_spec(),
                  _full_spec((D, D_FF)), _full_spec((1, D_FF)),
                  _full_spec((D_FF, D)), _full_spec((1, D)),
                  _full_spec((D, D_OUT)), _full_spec((1, D_OUT))],
        out_specs=_row_spec(D_OUT),
        out_shape=jax.ShapeDtypeStruct((N, D_OUT), jnp.float32),
    )(h, t, den, num, w1, b1, w2, b2, wc, bc)


_SC_MESH = plsc.VectorSubcoreMesh(
    core_axis_name="c", subcore_axis_name="s", num_cores=NC, num_subcores=NS)


def _sc_agg_body(em, p, srcs, dsts, zeros, den, num,
                 si0, si1, di0, di1, b0, b1, acc,
                 gs0, gs1, ss0, ss1, is0, is1):
    bufs = (b0, b1)
    gsems = (gs0, gs1)
    ssems = (ss0, ss1)
    sidx = (si0, si1)
    didx = (di0, di1)
    isems = (is0, is1)
    cid = lax.axis_index("c")
    sid = lax.axis_index("s")
    ngrp = CHUNKS // IGRP

    def run(table, out):
        pltpu.sync_copy(zeros.at[pl.ds(sid * ROWS_PER_SUB, ROWS_PER_SUB)],
                        acc.at[pl.ds(sid * ROWS_PER_SUB, ROWS_PER_SUB)])
        src_rows = srcs.at[sid]
        dst_rows = dsts.at[sid]

        def idx_copies(g, slot):
            off = pl.multiple_of(g * IGRP, IGRP)
            return (pltpu.make_async_copy(src_rows.at[pl.ds(off, IGRP)],
                                          sidx[slot], isems[slot]),
                    pltpu.make_async_copy(dst_rows.at[pl.ds(off, IGRP)],
                                          didx[slot], isems[slot]))

        for c in idx_copies(0, 0):
            c.start()
            c.wait()
        plsc.subcore_barrier()
        pltpu.async_copy(table.at[si0.at[0]], b0, gs0)

        def gather(idx_row, b):
            pltpu.async_copy(table.at[idx_row], bufs[b], gsems[b])

        def wait_gather(idx_row, b):
            pltpu.make_async_copy(table.at[idx_row], bufs[b], gsems[b]).wait()

        def scatter(idx_row, b):
            pltpu.async_copy(bufs[b], acc.at[idx_row], ssems[b], add=True)

        def wait_scatter(b):
            pltpu.make_async_copy(bufs[b], acc.at[didx[0].at[0]],
                                  ssems[b]).wait()

        def grp(g, slot):
            sV, dV = sidx[slot], didx[slot]
            for j0 in range(IGRP):
                b = j0 % 2
                nb = 1 - b
                wait_gather(sV.at[j0], b)
                scatter(dV.at[j0], b)
                if j0 + 1 < IGRP:
                    if j0 == 0:
                        @pl.when(g > 0)
                        def _():
                            wait_scatter(nb)
                    else:
                        wait_scatter(nb)
                    gather(sV.at[j0 + 1], nb)
                    if j0 == 0:
                        @pl.when(g + 1 < ngrp)
                        def _():
                            for c in idx_copies(g + 1, 1 - slot):
                                c.start()
                else:
                    @pl.when(g + 1 < ngrp)
                    def _():
                        for c in idx_copies(g + 1, 1 - slot):
                            c.wait()
                        wait_scatter(nb)
                        gather(sidx[1 - slot].at[0], nb)

        def grp2(g2, carry):
            grp(2 * g2, 0)
            grp(2 * g2 + 1, 1)
            return carry

        lax.fori_loop(0, ngrp // 2, grp2, 0, unroll=False)
        wait_scatter(0)
        wait_scatter(1)

        plsc.subcore_barrier()
        pltpu.sync_copy(
            acc.at[pl.ds(sid * OUT_ROWS_PER_SUB, OUT_ROWS_PER_SUB)],
            out.at[pl.ds(sid * OUT_ROWS_PER_SUB, OUT_ROWS_PER_SUB)])

        @pl.when(sid == NS - 1)
        def _():
            tail = NS * OUT_ROWS_PER_SUB
            pltpu.sync_copy(acc.at[pl.ds(tail, N - tail)],
                            out.at[pl.ds(tail, N - tail)])

    @pl.when(cid == 0)
    def _():
        run(em, den)

    @pl.when(cid == 1)
    def _():
        run(p, num)


_sc_agg = pl.kernel(
    _sc_agg_body,
    out_type=(jax.ShapeDtypeStruct((N, D), jnp.float32),
              jax.ShapeDtypeStruct((N, D), jnp.float32)),
    mesh=_SC_MESH,
    scratch_types=[
        pltpu.VMEM((IGRP, CHUNK), jnp.int32),
        pltpu.VMEM((IGRP, CHUNK), jnp.int32),
        pltpu.VMEM((IGRP, CHUNK), jnp.int32),
        pltpu.VMEM((IGRP, CHUNK), jnp.int32),
        pltpu.VMEM((CHUNK, D), jnp.float32),
        pltpu.VMEM((CHUNK, D), jnp.float32),
        pltpu.VMEM_SHARED((NPAD, D), jnp.float32),
        pltpu.SemaphoreType.DMA,
        pltpu.SemaphoreType.DMA,
        pltpu.SemaphoreType.DMA,
        pltpu.SemaphoreType.DMA,
        pltpu.SemaphoreType.DMA,
        pltpu.SemaphoreType.DMA,
    ],
)


def kernel(x, edge_index, W_enc, b_enc, ln_w1, ln_b1, Wm1_1, bm1_1, Wm2_1,
           bm2_1, ln_w2, ln_b2, Wm1_2, bm1_2, Wm2_2, bm2_2, ln_w3, ln_b3,
           Wm1_3, bm1_3, Wm2_3, bm2_3, W_cls, b_cls):
    f32 = jnp.float32
    src = edge_index[0]
    dst = edge_index[1]
    pad = EPAD - E
    srcp = jnp.concatenate([src, jnp.zeros((pad,), jnp.int32)])
    dstp = jnp.concatenate(
        [dst, N + (jnp.arange(pad, dtype=jnp.int32) % (NPAD - N))])
    srcs = srcp.reshape(NS, CHUNKS, CHUNK)
    dsts = dstp.reshape(NS, CHUNKS, CHUNK)
    zeros = jnp.zeros((NPAD, D), f32)

    r = lambda v: v.reshape(1, -1).astype(f32)

    h, t, em, p = _enc_pre(x, W_enc, r(b_enc), r(ln_w1), r(ln_b1))
    den, num = _sc_agg(em, p, srcs, dsts, zeros)
    h, t, em, p = _post_pre(h, t, den, num, Wm1_1, r(bm1_1), Wm2_1, r(bm2_1),
                            r(ln_w2), r(ln_b2))
    den, num = _sc_agg(em, p, srcs, dsts, zeros)
    h, t, em, p = _post_pre(h, t, den, num, Wm1_2, r(bm1_2), Wm2_2, r(bm2_2),
                            r(ln_w3), r(ln_b3))
    den, num = _sc_agg(em, p, srcs, dsts, zeros)
    return _post_cls(h, t, den, num, Wm1_3, r(bm1_3), Wm2_3, r(bm2_3),
                     W_cls, r(b_cls))


# speedup vs baseline: 1.0231x; 1.0231x over previous
import functools

import jax
import jax.numpy as jnp
from jax import lax
from jax.experimental import pallas as pl
from jax.experimental.pallas import tpu as pltpu
from jax.experimental.pallas import tpu_sc as plsc

N = 10000
E = 320000
D = 128
D_FF = 256
D_OUT = 112
EPS = 1e-7

NC = 2
NS = 16
CHUNK = 128
CHUNKS = 160
IGRP = 8
EPW = CHUNK * CHUNKS
EPAD = NS * EPW
NPAD = 10112
ROWS_PER_SUB = NPAD // NS
OUT_ROWS_PER_SUB = 624
BN = 1000
GRID = N // BN


def _ln_relu(h, w, b):
    mu = jnp.mean(h, axis=-1, keepdims=True)
    var = jnp.mean((h - mu) ** 2, axis=-1, keepdims=True)
    return jax.nn.relu((h - mu) * lax.rsqrt(var + 1e-5) * w + b)


def _tables(t):
    m = t + EPS
    em = jnp.exp(m)
    return em, em * m


def _enc_pre_body(x_ref, w_ref, b_ref, lw_ref, lb_ref,
                  h_ref, t_ref, em_ref, p_ref):
    h = jnp.dot(x_ref[...], w_ref[...], preferred_element_type=jnp.float32)
    h = h + b_ref[...]
    h_ref[...] = h
    t = _ln_relu(h, lw_ref[...], lb_ref[...])
    t_ref[...] = t
    em, p = _tables(t)
    em_ref[...] = em
    p_ref[...] = p


def _post_body(h_ref, t_ref, den_ref, num_ref, w1_ref, b1_ref, w2_ref, b2_ref):
    agg = num_ref[...] / (den_ref[...] + 1e-16)
    out = agg + t_ref[...]
    h1 = jax.nn.relu(
        jnp.dot(out, w1_ref[...], preferred_element_type=jnp.float32)
        + b1_ref[...])
    conv = jnp.dot(h1, w2_ref[...], preferred_element_type=jnp.float32)
    return h_ref[...] + conv + b2_ref[...]


def _post_pre_body(h_ref, t_ref, den_ref, num_ref, w1_ref, b1_ref, w2_ref,
                   b2_ref, lw_ref, lb_ref, hn_ref, tn_ref, em_ref, p_ref):
    hn = _post_body(h_ref, t_ref, den_ref, num_ref, w1_ref, b1_ref, w2_ref,
                    b2_ref)
    hn_ref[...] = hn
    tn = _ln_relu(hn, lw_ref[...], lb_ref[...])
    tn_ref[...] = tn
    em, p = _tables(tn)
    em_ref[...] = em
    p_ref[...] = p


def _post_cls_body(h_ref, t_ref, den_ref, num_ref, w1_ref, b1_ref, w2_ref,
                   b2_ref, wc_ref, bc_ref, o_ref):
    hn = _post_body(h_ref, t_ref, den_ref, num_ref, w1_ref, b1_ref, w2_ref,
                    b2_ref)
    o_ref[...] = (jnp.dot(hn, wc_ref[...], preferred_element_type=jnp.float32)
                  + bc_ref[...])


def _row_spec(d=D):
    return pl.BlockSpec((BN, d), lambda i: (i, 0))


def _full_spec(shape):
    return pl.BlockSpec(shape, lambda i: tuple(0 for _ in shape))


def _enc_pre(x, w, b, lw, lb):
    f32 = jnp.float32
    return pl.pallas_call(
        _enc_pre_body,
        grid=(GRID,),
        in_specs=[_row_spec(), _full_spec((D, D)), _full_spec((1, D)),
                  _full_spec((1, D)), _full_spec((1, D))],
        out_specs=[_row_spec(), _row_spec(), _row_spec(), _row_spec()],
        out_shape=[jax.ShapeDtypeStruct((N, D), f32)] * 4,
    )(x, w, b, lw, lb)


def _post_pre(h, t, den, num, w1, b1, w2, b2, lw, lb):
    f32 = jnp.float32
    return pl.pallas_call(
        _post_pre_body,
        grid=(GRID,),
        in_specs=[_row_spec(), _row_spec(), _row_spec(), _row_spec(),
                  _full_spec((D, D_FF)), _full_spec((1, D_FF)),
                  _full_spec((D_FF, D)), _full_spec((1, D)),
                  _full_spec((1, D)), _full_spec((1, D))],
        out_specs=[_row_spec(), _row_spec(), _row_spec(), _row_spec()],
        out_shape=[jax.ShapeDtypeStruct((N, D), f32)] * 4,
    )(h, t, den, num, w1, b1, w2, b2, lw, lb)


def _post_cls(h, t, den, num, w1, b1, w2, b2, wc, bc):
    return pl.pallas_call(
        _post_cls_body,
        grid=(GRID,),
        in_specs=[_row_spec(), _row_spec(), _row_spec(), _row_spec(),
                  _full_spec((D, D_FF)), _full_spec((1, D_FF)),
                  _full_spec((D_FF, D)), _full_spec((1, D)),
                  _full_spec((D, D_OUT)), _full_spec((1, D_OUT))],
        out_specs=_row_spec(D_OUT),
        out_shape=jax.ShapeDtypeStruct((N, D_OUT), jnp.float32),
    )(h, t, den, num, w1, b1, w2, b2, wc, bc)


_SC_MESH = plsc.VectorSubcoreMesh(
    core_axis_name="c", subcore_axis_name="s", num_cores=NC, num_subcores=NS)


def _sc_agg_body(em, p, srcs, dsts, zeros, den, num,
                 si0, si1, di0, di1, b0, b1, acc,
                 gs0, gs1, ss0, ss1, is0, is1):
    bufs = (b0, b1)
    gsems = (gs0, gs1)
    ssems = (ss0, ss1)
    sidx = (si0, si1)
    didx = (di0, di1)
    isems = (is0, is1)
    cid = lax.axis_index("c")
    sid = lax.axis_index("s")
    ngrp = CHUNKS // IGRP

    def run(table, out):
        pltpu.sync_copy(zeros.at[pl.ds(sid * ROWS_PER_SUB, ROWS_PER_SUB)],
                        acc.at[pl.ds(sid * ROWS_PER_SUB, ROWS_PER_SUB)])
        src_rows = srcs.at[sid]
        dst_rows = dsts.at[sid]

        def idx_copies(g, slot):
            off = pl.multiple_of(g * IGRP, IGRP)
            return (pltpu.make_async_copy(src_rows.at[pl.ds(off, IGRP)],
                                          sidx[slot], isems[slot]),
                    pltpu.make_async_copy(dst_rows.at[pl.ds(off, IGRP)],
                                          didx[slot], isems[slot]))

        for c in idx_copies(0, 0):
            c.start()
            c.wait()
        plsc.subcore_barrier()

        def gather(slot, j0, b):
            pltpu.async_copy(table.at[sidx[slot].at[j0]], bufs[b], gsems[b])

        def grp(g, slot):
            @pl.when(g > 0)
            def _():
                for c in idx_copies(g, slot):
                    c.wait()

            @pl.when(g + 1 < ngrp)
            def _():
                for c in idx_copies(g + 1, 1 - slot):
                    c.start()

            gather(slot, 0, 0)
            gather(slot, 1, 1)
            for j0 in range(IGRP):
                b = j0 % 2
                pltpu.make_async_copy(table.at[sidx[slot].at[j0]], bufs[b],
                                      gsems[b]).wait()
                pltpu.sync_copy(bufs[b], acc.at[didx[slot].at[j0]], add=True)
                if j0 + 2 < IGRP:
                    gather(slot, j0 + 2, b)

        def grp2(g2, carry):
            grp(2 * g2, 0)
            grp(2 * g2 + 1, 1)
            return carry

        lax.fori_loop(0, ngrp // 2, grp2, 0, unroll=False)

        plsc.subcore_barrier()
        pltpu.sync_copy(
            acc.at[pl.ds(sid * OUT_ROWS_PER_SUB, OUT_ROWS_PER_SUB)],
            out.at[pl.ds(sid * OUT_ROWS_PER_SUB, OUT_ROWS_PER_SUB)])

        @pl.when(sid == NS - 1)
        def _():
            tail = NS * OUT_ROWS_PER_SUB
            pltpu.sync_copy(acc.at[pl.ds(tail, N - tail)],
                            out.at[pl.ds(tail, N - tail)])

    @pl.when(cid == 0)
    def _():
        run(em, den)

    @pl.when(cid == 1)
    def _():
        run(p, num)


_sc_agg = pl.kernel(
    _sc_agg_body,
    out_type=(jax.ShapeDtypeStruct((N, D), jnp.float32),
              jax.ShapeDtypeStruct((N, D), jnp.float32)),
    mesh=_SC_MESH,
    scratch_types=[
        pltpu.VMEM((IGRP, CHUNK), jnp.int32),
        pltpu.VMEM((IGRP, CHUNK), jnp.int32),
        pltpu.VMEM((IGRP, CHUNK), jnp.int32),
        pltpu.VMEM((IGRP, CHUNK), jnp.int32),
        pltpu.VMEM((CHUNK, D), jnp.float32),
        pltpu.VMEM((CHUNK, D), jnp.float32),
        pltpu.VMEM_SHARED((NPAD, D), jnp.float32),
        pltpu.SemaphoreType.DMA,
        pltpu.SemaphoreType.DMA,
        pltpu.SemaphoreType.DMA,
        pltpu.SemaphoreType.DMA,
        pltpu.SemaphoreType.DMA,
        pltpu.SemaphoreType.DMA,
    ],
)


def kernel(x, edge_index, W_enc, b_enc, ln_w1, ln_b1, Wm1_1, bm1_1, Wm2_1,
           bm2_1, ln_w2, ln_b2, Wm1_2, bm1_2, Wm2_2, bm2_2, ln_w3, ln_b3,
           Wm1_3, bm1_3, Wm2_3, bm2_3, W_cls, b_cls):
    f32 = jnp.float32
    src = edge_index[0]
    dst = edge_index[1]
    pad = EPAD - E
    srcp = jnp.concatenate([src, jnp.zeros((pad,), jnp.int32)])
    dstp = jnp.concatenate(
        [dst, N + (jnp.arange(pad, dtype=jnp.int32) % (NPAD - N))])
    srcs = srcp.reshape(NS, CHUNKS, CHUNK)
    dsts = dstp.reshape(NS, CHUNKS, CHUNK)
    zeros = jnp.zeros((NPAD, D), f32)

    r = lambda v: v.reshape(1, -1).astype(f32)

    h, t, em, p = _enc_pre(x, W_enc, r(b_enc), r(ln_w1), r(ln_b1))
    den, num = _sc_agg(em, p, srcs, dsts, zeros)
    h, t, em, p = _post_pre(h, t, den, num, Wm1_1, r(bm1_1), Wm2_1, r(bm2_1),
                            r(ln_w2), r(ln_b2))
    den, num = _sc_agg(em, p, srcs, dsts, zeros)
    h, t, em, p = _post_pre(h, t, den, num, Wm1_2, r(bm1_2), Wm2_2, r(bm2_2),
                            r(ln_w3), r(ln_b3))
    den, num = _sc_agg(em, p, srcs, dsts, zeros)
    return _post_cls(h, t, den, num, Wm1_3, r(bm1_3), Wm2_3, r(bm2_3),
                     W_cls, r(b_cls))


# speedup vs baseline: 1.0446x; 1.0210x over previous
import functools

import jax
import jax.numpy as jnp
from jax import lax
from jax.experimental import pallas as pl
from jax.experimental.pallas import tpu as pltpu
from jax.experimental.pallas import tpu_sc as plsc

N = 10000
E = 320000
D = 128
D_FF = 256
D_OUT = 112
EPS = 1e-7

NC = 2
NS = 16
CHUNK = 128
CHUNKS = 160
IGRP = 16
EPW = CHUNK * CHUNKS
EPAD = NS * EPW
NPAD = 10112
ROWS_PER_SUB = NPAD // NS
OUT_ROWS_PER_SUB = 624
BN = 1000
GRID = N // BN


def _ln_relu(h, w, b):
    mu = jnp.mean(h, axis=-1, keepdims=True)
    var = jnp.mean((h - mu) ** 2, axis=-1, keepdims=True)
    return jax.nn.relu((h - mu) * lax.rsqrt(var + 1e-5) * w + b)


def _tables(t):
    m = t + EPS
    em = jnp.exp(m)
    return em, em * m


def _enc_pre_body(x_ref, w_ref, b_ref, lw_ref, lb_ref,
                  h_ref, t_ref, em_ref, p_ref):
    h = jnp.dot(x_ref[...], w_ref[...], preferred_element_type=jnp.float32)
    h = h + b_ref[...]
    h_ref[...] = h
    t = _ln_relu(h, lw_ref[...], lb_ref[...])
    t_ref[...] = t
    em, p = _tables(t)
    em_ref[...] = em
    p_ref[...] = p


def _post_body(h_ref, t_ref, den_ref, num_ref, w1_ref, b1_ref, w2_ref, b2_ref):
    agg = num_ref[...] / (den_ref[...] + 1e-16)
    out = agg + t_ref[...]
    h1 = jax.nn.relu(
        jnp.dot(out, w1_ref[...], preferred_element_type=jnp.float32)
        + b1_ref[...])
    conv = jnp.dot(h1, w2_ref[...], preferred_element_type=jnp.float32)
    return h_ref[...] + conv + b2_ref[...]


def _post_pre_body(h_ref, t_ref, den_ref, num_ref, w1_ref, b1_ref, w2_ref,
                   b2_ref, lw_ref, lb_ref, hn_ref, tn_ref, em_ref, p_ref):
    hn = _post_body(h_ref, t_ref, den_ref, num_ref, w1_ref, b1_ref, w2_ref,
                    b2_ref)
    hn_ref[...] = hn
    tn = _ln_relu(hn, lw_ref[...], lb_ref[...])
    tn_ref[...] = tn
    em, p = _tables(tn)
    em_ref[...] = em
    p_ref[...] = p


def _post_cls_body(h_ref, t_ref, den_ref, num_ref, w1_ref, b1_ref, w2_ref,
                   b2_ref, wc_ref, bc_ref, o_ref):
    hn = _post_body(h_ref, t_ref, den_ref, num_ref, w1_ref, b1_ref, w2_ref,
                    b2_ref)
    o_ref[...] = (jnp.dot(hn, wc_ref[...], preferred_element_type=jnp.float32)
                  + bc_ref[...])


def _row_spec(d=D):
    return pl.BlockSpec((BN, d), lambda i: (i, 0))


def _full_spec(shape):
    return pl.BlockSpec(shape, lambda i: tuple(0 for _ in shape))


def _enc_pre(x, w, b, lw, lb):
    f32 = jnp.float32
    return pl.pallas_call(
        _enc_pre_body,
        grid=(GRID,),
        in_specs=[_row_spec(), _full_spec((D, D)), _full_spec((1, D)),
                  _full_spec((1, D)), _full_spec((1, D))],
        out_specs=[_row_spec(), _row_spec(), _row_spec(), _row_spec()],
        out_shape=[jax.ShapeDtypeStruct((N, D), f32)] * 4,
    )(x, w, b, lw, lb)


def _post_pre(h, t, den, num, w1, b1, w2, b2, lw, lb):
    f32 = jnp.float32
    return pl.pallas_call(
        _post_pre_body,
        grid=(GRID,),
        in_specs=[_row_spec(), _row_spec(), _row_spec(), _row_spec(),
                  _full_spec((D, D_FF)), _full_spec((1, D_FF)),
                  _full_spec((D_FF, D)), _full_spec((1, D)),
                  _full_spec((1, D)), _full_spec((1, D))],
        out_specs=[_row_spec(), _row_spec(), _row_spec(), _row_spec()],
        out_shape=[jax.ShapeDtypeStruct((N, D), f32)] * 4,
    )(h, t, den, num, w1, b1, w2, b2, lw, lb)


def _post_cls(h, t, den, num, w1, b1, w2, b2, wc, bc):
    return pl.pallas_call(
        _post_cls_body,
        grid=(GRID,),
        in_specs=[_row_spec(), _row_spec(), _row_spec(), _row_spec(),
                  _full_spec((D, D_FF)), _full_spec((1, D_FF)),
                  _full_spec((D_FF, D)), _full_spec((1, D)),
                  _full_spec((D, D_OUT)), _full_spec((1, D_OUT))],
        out_specs=_row_spec(D_OUT),
        out_shape=jax.ShapeDtypeStruct((N, D_OUT), jnp.float32),
    )(h, t, den, num, w1, b1, w2, b2, wc, bc)


_SC_MESH = plsc.VectorSubcoreMesh(
    core_axis_name="c", subcore_axis_name="s", num_cores=NC, num_subcores=NS)


def _sc_agg_body(em, p, srcs, dsts, zeros, den, num,
                 si0, si1, di0, di1, b0, b1, acc,
                 gs0, gs1, ss0, ss1, is0, is1):
    bufs = (b0, b1)
    gsems = (gs0, gs1)
    ssems = (ss0, ss1)
    sidx = (si0, si1)
    didx = (di0, di1)
    isems = (is0, is1)
    cid = lax.axis_index("c")
    sid = lax.axis_index("s")
    ngrp = CHUNKS // IGRP

    def run(table, out):
        pltpu.sync_copy(zeros.at[pl.ds(sid * ROWS_PER_SUB, ROWS_PER_SUB)],
                        acc.at[pl.ds(sid * ROWS_PER_SUB, ROWS_PER_SUB)])
        src_rows = srcs.at[sid]
        dst_rows = dsts.at[sid]

        def idx_copies(g, slot):
            off = pl.multiple_of(g * IGRP, IGRP)
            return (pltpu.make_async_copy(src_rows.at[pl.ds(off, IGRP)],
                                          sidx[slot], isems[slot]),
                    pltpu.make_async_copy(dst_rows.at[pl.ds(off, IGRP)],
                                          didx[slot], isems[slot]))

        for c in idx_copies(0, 0):
            c.start()
            c.wait()
        plsc.subcore_barrier()

        def gather(slot, j0, b):
            pltpu.async_copy(table.at[sidx[slot].at[j0]], bufs[b], gsems[b])

        def grp(g, slot):
            @pl.when(g > 0)
            def _():
                for c in idx_copies(g, slot):
                    c.wait()

            @pl.when(g + 1 < ngrp)
            def _():
                for c in idx_copies(g + 1, 1 - slot):
                    c.start()

            gather(slot, 0, 0)
            gather(slot, 1, 1)
            for j0 in range(IGRP):
                b = j0 % 2
                pltpu.make_async_copy(table.at[sidx[slot].at[j0]], bufs[b],
                                      gsems[b]).wait()
                pltpu.sync_copy(bufs[b], acc.at[didx[slot].at[j0]], add=True)
                if j0 + 2 < IGRP:
                    gather(slot, j0 + 2, b)

        def grp2(g2, carry):
            grp(2 * g2, 0)
            grp(2 * g2 + 1, 1)
            return carry

        lax.fori_loop(0, ngrp // 2, grp2, 0, unroll=False)

        plsc.subcore_barrier()
        pltpu.sync_copy(
            acc.at[pl.ds(sid * OUT_ROWS_PER_SUB, OUT_ROWS_PER_SUB)],
            out.at[pl.ds(sid * OUT_ROWS_PER_SUB, OUT_ROWS_PER_SUB)])

        @pl.when(sid == NS - 1)
        def _():
            tail = NS * OUT_ROWS_PER_SUB
            pltpu.sync_copy(acc.at[pl.ds(tail, N - tail)],
                            out.at[pl.ds(tail, N - tail)])

    @pl.when(cid == 0)
    def _():
        run(em, den)

    @pl.when(cid == 1)
    def _():
        run(p, num)


_sc_agg = pl.kernel(
    _sc_agg_body,
    out_type=(jax.ShapeDtypeStruct((N, D), jnp.float32),
              jax.ShapeDtypeStruct((N, D), jnp.float32)),
    mesh=_SC_MESH,
    scratch_types=[
        pltpu.VMEM((IGRP, CHUNK), jnp.int32),
        pltpu.VMEM((IGRP, CHUNK), jnp.int32),
        pltpu.VMEM((IGRP, CHUNK), jnp.int32),
        pltpu.VMEM((IGRP, CHUNK), jnp.int32),
        pltpu.VMEM((CHUNK, D), jnp.float32),
        pltpu.VMEM((CHUNK, D), jnp.float32),
        pltpu.VMEM_SHARED((NPAD, D), jnp.float32),
        pltpu.SemaphoreType.DMA,
        pltpu.SemaphoreType.DMA,
        pltpu.SemaphoreType.DMA,
        pltpu.SemaphoreType.DMA,
        pltpu.SemaphoreType.DMA,
        pltpu.SemaphoreType.DMA,
    ],
)


def kernel(x, edge_index, W_enc, b_enc, ln_w1, ln_b1, Wm1_1, bm1_1, Wm2_1,
           bm2_1, ln_w2, ln_b2, Wm1_2, bm1_2, Wm2_2, bm2_2, ln_w3, ln_b3,
           Wm1_3, bm1_3, Wm2_3, bm2_3, W_cls, b_cls):
    f32 = jnp.float32
    src = edge_index[0]
    dst = edge_index[1]
    pad = EPAD - E
    srcp = jnp.concatenate([src, jnp.zeros((pad,), jnp.int32)])
    dstp = jnp.concatenate(
        [dst, N + (jnp.arange(pad, dtype=jnp.int32) % (NPAD - N))])
    srcs = srcp.reshape(NS, CHUNKS, CHUNK)
    dsts = dstp.reshape(NS, CHUNKS, CHUNK)
    zeros = jnp.zeros((NPAD, D), f32)

    r = lambda v: v.reshape(1, -1).astype(f32)

    h, t, em, p = _enc_pre(x, W_enc, r(b_enc), r(ln_w1), r(ln_b1))
    den, num = _sc_agg(em, p, srcs, dsts, zeros)
    h, t, em, p = _post_pre(h, t, den, num, Wm1_1, r(bm1_1), Wm2_1, r(bm2_1),
                            r(ln_w2), r(ln_b2))
    den, num = _sc_agg(em, p, srcs, dsts, zeros)
    h, t, em, p = _post_pre(h, t, den, num, Wm1_2, r(bm1_2), Wm2_2, r(bm2_2),
                            r(ln_w3), r(ln_b3))
    den, num = _sc_agg(em, p, srcs, dsts, zeros)
    return _post_cls(h, t, den, num, Wm1_3, r(bm1_3), Wm2_3, r(bm2_3),
                     W_cls, r(b_cls))


# speedup vs baseline: 1.0690x; 1.0234x over previous
import functools

import jax
import jax.numpy as jnp
from jax import lax
from jax.experimental import pallas as pl
from jax.experimental.pallas import tpu as pltpu
from jax.experimental.pallas import tpu_sc as plsc

N = 10000
E = 320000
D = 128
D_FF = 256
D_OUT = 112
EPS = 1e-7

NC = 2
NS = 16
CHUNK = 128
CHUNKS = 160
IGRP = 16
EPW = CHUNK * CHUNKS
EPAD = NS * EPW
NPAD = 10112
ROWS_PER_SUB = NPAD // NS
OUT_ROWS_PER_SUB = 624
BN = 1000
GRID = N // BN


def _ln_relu(h, w, b):
    mu = jnp.mean(h, axis=-1, keepdims=True)
    var = jnp.mean((h - mu) ** 2, axis=-1, keepdims=True)
    return jax.nn.relu((h - mu) * lax.rsqrt(var + 1e-5) * w + b)


def _tables(t):
    m = t + EPS
    em = jnp.exp(m)
    return em, em * m


def _enc_pre_body(x_ref, w_ref, b_ref, lw_ref, lb_ref,
                  h_ref, t_ref, em_ref, p_ref):
    h = jnp.dot(x_ref[...], w_ref[...], preferred_element_type=jnp.float32)
    h = h + b_ref[...]
    h_ref[...] = h
    t = _ln_relu(h, lw_ref[...], lb_ref[...])
    t_ref[...] = t
    em, p = _tables(t)
    em_ref[...] = em
    p_ref[...] = p


def _post_body(h_ref, t_ref, den_ref, num_ref, w1_ref, b1_ref, w2_ref, b2_ref):
    agg = num_ref[...] / (den_ref[...] + 1e-16)
    out = agg + t_ref[...]
    h1 = jax.nn.relu(
        jnp.dot(out, w1_ref[...], preferred_element_type=jnp.float32)
        + b1_ref[...])
    conv = jnp.dot(h1, w2_ref[...], preferred_element_type=jnp.float32)
    return h_ref[...] + conv + b2_ref[...]


def _post_pre_body(h_ref, t_ref, den_ref, num_ref, w1_ref, b1_ref, w2_ref,
                   b2_ref, lw_ref, lb_ref, hn_ref, tn_ref, em_ref, p_ref):
    hn = _post_body(h_ref, t_ref, den_ref, num_ref, w1_ref, b1_ref, w2_ref,
                    b2_ref)
    hn_ref[...] = hn
    tn = _ln_relu(hn, lw_ref[...], lb_ref[...])
    tn_ref[...] = tn
    em, p = _tables(tn)
    em_ref[...] = em
    p_ref[...] = p


def _post_cls_body(h_ref, t_ref, den_ref, num_ref, w1_ref, b1_ref, w2_ref,
                   b2_ref, wc_ref, bc_ref, o_ref):
    hn = _post_body(h_ref, t_ref, den_ref, num_ref, w1_ref, b1_ref, w2_ref,
                    b2_ref)
    o_ref[...] = (jnp.dot(hn, wc_ref[...], preferred_element_type=jnp.float32)
                  + bc_ref[...])


def _row_spec(d=D):
    return pl.BlockSpec((BN, d), lambda i: (i, 0))


def _full_spec(shape):
    return pl.BlockSpec(shape, lambda i: tuple(0 for _ in shape))


def _enc_pre(x, w, b, lw, lb):
    f32 = jnp.float32
    return pl.pallas_call(
        _enc_pre_body,
        grid=(GRID,),
        in_specs=[_row_spec(), _full_spec((D, D)), _full_spec((1, D)),
                  _full_spec((1, D)), _full_spec((1, D))],
        out_specs=[_row_spec(), _row_spec(), _row_spec(), _row_spec()],
        out_shape=[jax.ShapeDtypeStruct((N, D), f32)] * 4,
    )(x, w, b, lw, lb)


def _post_pre(h, t, den, num, w1, b1, w2, b2, lw, lb):
    f32 = jnp.float32
    return pl.pallas_call(
        _post_pre_body,
        grid=(GRID,),
        in_specs=[_row_spec(), _row_spec(), _row_spec(), _row_spec(),
                  _full_spec((D, D_FF)), _full_spec((1, D_FF)),
                  _full_spec((D_FF, D)), _full_spec((1, D)),
                  _full_spec((1, D)), _full_spec((1, D))],
        out_specs=[_row_spec(), _row_spec(), _row_spec(), _row_spec()],
        out_shape=[jax.ShapeDtypeStruct((N, D), f32)] * 4,
    )(h, t, den, num, w1, b1, w2, b2, lw, lb)


def _post_cls(h, t, den, num, w1, b1, w2, b2, wc, bc):
    return pl.pallas_call(
        _post_cls_body,
        grid=(GRID,),
        in_specs=[_row_spec(), _row_spec(), _row_spec(), _row_spec(),
                  _full_spec((D, D_FF)), _full_spec((1, D_FF)),
                  _full_spec((D_FF, D)), _full_spec((1, D)),
                  _full_spec((D, D_OUT)), _full_spec((1, D_OUT))],
        out_specs=_row_spec(D_OUT),
        out_shape=jax.ShapeDtypeStruct((N, D_OUT), jnp.float32),
    )(h, t, den, num, w1, b1, w2, b2, wc, bc)


_SC_MESH = plsc.VectorSubcoreMesh(
    core_axis_name="c", subcore_axis_name="s", num_cores=NC, num_subcores=NS)


def _sc_agg_body(em, p, srcs, dsts, zeros, den, num,
                 si0, si1, di0, di1, b0, b1, acc,
                 gs0, gs1, ss0, ss1, is0, is1):
    bufs = (b0, b1)
    gsems = (gs0, gs1)
    ssems = (ss0, ss1)
    sidx = (si0, si1)
    didx = (di0, di1)
    isems = (is0, is1)
    cid = lax.axis_index("c")
    sid = lax.axis_index("s")
    ngrp = CHUNKS // IGRP

    def run(table, out):
        pltpu.sync_copy(zeros.at[pl.ds(sid * ROWS_PER_SUB, ROWS_PER_SUB)],
                        acc.at[pl.ds(sid * ROWS_PER_SUB, ROWS_PER_SUB)])
        src_rows = srcs.at[sid]
        dst_rows = dsts.at[sid]

        def idx_copies(g, slot):
            off = pl.multiple_of(g * IGRP, IGRP)
            return (pltpu.make_async_copy(src_rows.at[pl.ds(off, IGRP)],
                                          sidx[slot], isems[slot]),
                    pltpu.make_async_copy(dst_rows.at[pl.ds(off, IGRP)],
                                          didx[slot], isems[slot]))

        for c in idx_copies(0, 0):
            c.start()
            c.wait()
        plsc.subcore_barrier()

        def gather(slot, j0, b):
            pltpu.async_copy(table.at[sidx[slot].at[j0]], bufs[b], gsems[b])

        gather(0, 0, 0)
        gather(0, 1, 1)

        def grp(g, slot):
            for j0 in range(IGRP):
                b = j0 % 2
                pltpu.make_async_copy(table.at[sidx[slot].at[j0]], bufs[b],
                                      gsems[b]).wait()
                pltpu.sync_copy(bufs[b], acc.at[didx[slot].at[j0]], add=True)
                if j0 + 2 < IGRP:
                    gather(slot, j0 + 2, b)
                elif j0 == IGRP - 2:
                    @pl.when(g + 1 < ngrp)
                    def _():
                        for c in idx_copies(g + 1, 1 - slot):
                            c.wait()
                        gather(1 - slot, 0, b)
                else:
                    @pl.when(g + 1 < ngrp)
                    def _():
                        gather(1 - slot, 1, b)
                if j0 == 0:
                    @pl.when(g + 1 < ngrp)
                    def _():
                        for c in idx_copies(g + 1, 1 - slot):
                            c.start()

        def grp2(g2, carry):
            grp(2 * g2, 0)
            grp(2 * g2 + 1, 1)
            return carry

        lax.fori_loop(0, ngrp // 2, grp2, 0, unroll=False)

        plsc.subcore_barrier()
        pltpu.sync_copy(
            acc.at[pl.ds(sid * OUT_ROWS_PER_SUB, OUT_ROWS_PER_SUB)],
            out.at[pl.ds(sid * OUT_ROWS_PER_SUB, OUT_ROWS_PER_SUB)])

        @pl.when(sid == NS - 1)
        def _():
            tail = NS * OUT_ROWS_PER_SUB
            pltpu.sync_copy(acc.at[pl.ds(tail, N - tail)],
                            out.at[pl.ds(tail, N - tail)])

    @pl.when(cid == 0)
    def _():
        run(em, den)

    @pl.when(cid == 1)
    def _():
        run(p, num)


_sc_agg = pl.kernel(
    _sc_agg_body,
    out_type=(jax.ShapeDtypeStruct((N, D), jnp.float32),
              jax.ShapeDtypeStruct((N, D), jnp.float32)),
    mesh=_SC_MESH,
    scratch_types=[
        pltpu.VMEM((IGRP, CHUNK), jnp.int32),
        pltpu.VMEM((IGRP, CHUNK), jnp.int32),
        pltpu.VMEM((IGRP, CHUNK), jnp.int32),
        pltpu.VMEM((IGRP, CHUNK), jnp.int32),
        pltpu.VMEM((CHUNK, D), jnp.float32),
        pltpu.VMEM((CHUNK, D), jnp.float32),
        pltpu.VMEM_SHARED((NPAD, D), jnp.float32),
        pltpu.SemaphoreType.DMA,
        pltpu.SemaphoreType.DMA,
        pltpu.SemaphoreType.DMA,
        pltpu.SemaphoreType.DMA,
        pltpu.SemaphoreType.DMA,
        pltpu.SemaphoreType.DMA,
    ],
)


def kernel(x, edge_index, W_enc, b_enc, ln_w1, ln_b1, Wm1_1, bm1_1, Wm2_1,
           bm2_1, ln_w2, ln_b2, Wm1_2, bm1_2, Wm2_2, bm2_2, ln_w3, ln_b3,
           Wm1_3, bm1_3, Wm2_3, bm2_3, W_cls, b_cls):
    f32 = jnp.float32
    src = edge_index[0]
    dst = edge_index[1]
    pad = EPAD - E
    srcp = jnp.concatenate([src, jnp.zeros((pad,), jnp.int32)])
    dstp = jnp.concatenate(
        [dst, N + (jnp.arange(pad, dtype=jnp.int32) % (NPAD - N))])
    srcs = srcp.reshape(NS, CHUNKS, CHUNK)
    dsts = dstp.reshape(NS, CHUNKS, CHUNK)
    zeros = jnp.zeros((NPAD, D), f32)

    r = lambda v: v.reshape(1, -1).astype(f32)

    h, t, em, p = _enc_pre(x, W_enc, r(b_enc), r(ln_w1), r(ln_b1))
    den, num = _sc_agg(em, p, srcs, dsts, zeros)
    h, t, em, p = _post_pre(h, t, den, num, Wm1_1, r(bm1_1), Wm2_1, r(bm2_1),
                            r(ln_w2), r(ln_b2))
    den, num = _sc_agg(em, p, srcs, dsts, zeros)
    h, t, em, p = _post_pre(h, t, den, num, Wm1_2, r(bm1_2), Wm2_2, r(bm2_2),
                            r(ln_w3), r(ln_b3))
    den, num = _sc_agg(em, p, srcs, dsts, zeros)
    return _post_cls(h, t, den, num, Wm1_3, r(bm1_3), Wm2_3, r(bm2_3),
                     W_cls, r(b_cls))


# speedup vs baseline: 1.0781x; 1.0086x over previous
import functools

import jax
import jax.numpy as jnp
from jax import lax
from jax.experimental import pallas as pl
from jax.experimental.pallas import tpu as pltpu
from jax.experimental.pallas import tpu_sc as plsc

N = 10000
E = 320000
D = 128
D_FF = 256
D_OUT = 112
EPS = 1e-7

NC = 2
NS = 16
CHUNK = 128
CHUNKS = 160
IGRP = 16
EPW = CHUNK * CHUNKS
EPAD = NS * EPW
NPAD = 10112
ROWS_PER_SUB = NPAD // NS
OUT_ROWS_PER_SUB = 624
BN = 2000
GRID = N // BN


def _ln_relu(h, w, b):
    mu = jnp.mean(h, axis=-1, keepdims=True)
    var = jnp.mean((h - mu) ** 2, axis=-1, keepdims=True)
    return jax.nn.relu((h - mu) * lax.rsqrt(var + 1e-5) * w + b)


def _tables(t):
    m = t + EPS
    em = jnp.exp(m)
    return em, em * m


def _enc_pre_body(x_ref, w_ref, b_ref, lw_ref, lb_ref,
                  h_ref, t_ref, em_ref, p_ref):
    h = jnp.dot(x_ref[...], w_ref[...], preferred_element_type=jnp.float32)
    h = h + b_ref[...]
    h_ref[...] = h
    t = _ln_relu(h, lw_ref[...], lb_ref[...])
    t_ref[...] = t
    em, p = _tables(t)
    em_ref[...] = em
    p_ref[...] = p


def _post_body(h_ref, t_ref, den_ref, num_ref, w1_ref, b1_ref, w2_ref, b2_ref):
    agg = num_ref[...] / (den_ref[...] + 1e-16)
    out = agg + t_ref[...]
    h1 = jax.nn.relu(
        jnp.dot(out, w1_ref[...], preferred_element_type=jnp.float32)
        + b1_ref[...])
    conv = jnp.dot(h1, w2_ref[...], preferred_element_type=jnp.float32)
    return h_ref[...] + conv + b2_ref[...]


def _post_pre_body(h_ref, t_ref, den_ref, num_ref, w1_ref, b1_ref, w2_ref,
                   b2_ref, lw_ref, lb_ref, hn_ref, tn_ref, em_ref, p_ref):
    hn = _post_body(h_ref, t_ref, den_ref, num_ref, w1_ref, b1_ref, w2_ref,
                    b2_ref)
    hn_ref[...] = hn
    tn = _ln_relu(hn, lw_ref[...], lb_ref[...])
    tn_ref[...] = tn
    em, p = _tables(tn)
    em_ref[...] = em
    p_ref[...] = p


def _post_cls_body(h_ref, t_ref, den_ref, num_ref, w1_ref, b1_ref, w2_ref,
                   b2_ref, wc_ref, bc_ref, o_ref):
    hn = _post_body(h_ref, t_ref, den_ref, num_ref, w1_ref, b1_ref, w2_ref,
                    b2_ref)
    o_ref[...] = (jnp.dot(hn, wc_ref[...], preferred_element_type=jnp.float32)
                  + bc_ref[...])


def _row_spec(d=D):
    return pl.BlockSpec((BN, d), lambda i: (i, 0))


def _full_spec(shape):
    return pl.BlockSpec(shape, lambda i: tuple(0 for _ in shape))


def _enc_pre(x, w, b, lw, lb):
    f32 = jnp.float32
    return pl.pallas_call(
        _enc_pre_body,
        grid=(GRID,),
        in_specs=[_row_spec(), _full_spec((D, D)), _full_spec((1, D)),
                  _full_spec((1, D)), _full_spec((1, D))],
        out_specs=[_row_spec(), _row_spec(), _row_spec(), _row_spec()],
        out_shape=[jax.ShapeDtypeStruct((N, D), f32)] * 4,
    )(x, w, b, lw, lb)


def _post_pre(h, t, den, num, w1, b1, w2, b2, lw, lb):
    f32 = jnp.float32
    return pl.pallas_call(
        _post_pre_body,
        grid=(GRID,),
        in_specs=[_row_spec(), _row_spec(), _row_spec(), _row_spec(),
                  _full_spec((D, D_FF)), _full_spec((1, D_FF)),
                  _full_spec((D_FF, D)), _full_spec((1, D)),
                  _full_spec((1, D)), _full_spec((1, D))],
        out_specs=[_row_spec(), _row_spec(), _row_spec(), _row_spec()],
        out_shape=[jax.ShapeDtypeStruct((N, D), f32)] * 4,
    )(h, t, den, num, w1, b1, w2, b2, lw, lb)


def _post_cls(h, t, den, num, w1, b1, w2, b2, wc, bc):
    return pl.pallas_call(
        _post_cls_body,
        grid=(GRID,),
        in_specs=[_row_spec(), _row_spec(), _row_spec(), _row_spec(),
                  _full_spec((D, D_FF)), _full_spec((1, D_FF)),
                  _full_spec((D_FF, D)), _full_spec((1, D)),
                  _full_spec((D, D_OUT)), _full_spec((1, D_OUT))],
        out_specs=_row_spec(D_OUT),
        out_shape=jax.ShapeDtypeStruct((N, D_OUT), jnp.float32),
    )(h, t, den, num, w1, b1, w2, b2, wc, bc)


_SC_MESH = plsc.VectorSubcoreMesh(
    core_axis_name="c", subcore_axis_name="s", num_cores=NC, num_subcores=NS)


def _sc_agg_body(em, p, srcs, dsts, zeros, den, num,
                 si0, si1, di0, di1, b0, b1, acc,
                 gs0, gs1, ss0, ss1, is0, is1):
    bufs = (b0, b1)
    gsems = (gs0, gs1)
    ssems = (ss0, ss1)
    sidx = (si0, si1)
    didx = (di0, di1)
    isems = (is0, is1)
    cid = lax.axis_index("c")
    sid = lax.axis_index("s")
    ngrp = CHUNKS // IGRP

    def run(table, out):
        zcp = pltpu.make_async_copy(
            zeros.at[pl.ds(sid * ROWS_PER_SUB, ROWS_PER_SUB)],
            acc.at[pl.ds(sid * ROWS_PER_SUB, ROWS_PER_SUB)], ss0)
        zcp.start()
        src_rows = srcs.at[sid]
        dst_rows = dsts.at[sid]

        def idx_copies(g, slot):
            off = pl.multiple_of(g * IGRP, IGRP)
            return (pltpu.make_async_copy(src_rows.at[pl.ds(off, IGRP)],
                                          sidx[slot], isems[slot]),
                    pltpu.make_async_copy(dst_rows.at[pl.ds(off, IGRP)],
                                          didx[slot], isems[slot]))

        primes = idx_copies(0, 0)
        for c in primes:
            c.start()
        for c in primes:
            c.wait()
        zcp.wait()
        plsc.subcore_barrier()

        def gather(slot, j0, b):
            pltpu.async_copy(table.at[sidx[slot].at[j0]], bufs[b], gsems[b])

        gather(0, 0, 0)
        gather(0, 1, 1)

        def grp(g, slot):
            for j0 in range(IGRP):
                b = j0 % 2
                pltpu.make_async_copy(table.at[sidx[slot].at[j0]], bufs[b],
                                      gsems[b]).wait()
                pltpu.sync_copy(bufs[b], acc.at[didx[slot].at[j0]], add=True)
                if j0 + 2 < IGRP:
                    gather(slot, j0 + 2, b)
                elif j0 == IGRP - 2:
                    @pl.when(g + 1 < ngrp)
                    def _():
                        for c in idx_copies(g + 1, 1 - slot):
                            c.wait()
                        gather(1 - slot, 0, b)
                else:
                    @pl.when(g + 1 < ngrp)
                    def _():
                        gather(1 - slot, 1, b)
                if j0 == 0:
                    @pl.when(g + 1 < ngrp)
                    def _():
                        for c in idx_copies(g + 1, 1 - slot):
                            c.start()

        def grp2(g2, carry):
            grp(2 * g2, 0)
            grp(2 * g2 + 1, 1)
            return carry

        lax.fori_loop(0, ngrp // 2, grp2, 0, unroll=False)

        plsc.subcore_barrier()
        pltpu.sync_copy(
            acc.at[pl.ds(sid * OUT_ROWS_PER_SUB, OUT_ROWS_PER_SUB)],
            out.at[pl.ds(sid * OUT_ROWS_PER_SUB, OUT_ROWS_PER_SUB)])

        @pl.when(sid == NS - 1)
        def _():
            tail = NS * OUT_ROWS_PER_SUB
            pltpu.sync_copy(acc.at[pl.ds(tail, N - tail)],
                            out.at[pl.ds(tail, N - tail)])

    @pl.when(cid == 0)
    def _():
        run(em, den)

    @pl.when(cid == 1)
    def _():
        run(p, num)


_sc_agg = pl.kernel(
    _sc_agg_body,
    out_type=(jax.ShapeDtypeStruct((N, D), jnp.float32),
              jax.ShapeDtypeStruct((N, D), jnp.float32)),
    mesh=_SC_MESH,
    scratch_types=[
        pltpu.VMEM((IGRP, CHUNK), jnp.int32),
        pltpu.VMEM((IGRP, CHUNK), jnp.int32),
        pltpu.VMEM((IGRP, CHUNK), jnp.int32),
        pltpu.VMEM((IGRP, CHUNK), jnp.int32),
        pltpu.VMEM((CHUNK, D), jnp.float32),
        pltpu.VMEM((CHUNK, D), jnp.float32),
        pltpu.VMEM_SHARED((NPAD, D), jnp.float32),
        pltpu.SemaphoreType.DMA,
        pltpu.SemaphoreType.DMA,
        pltpu.SemaphoreType.DMA,
        pltpu.SemaphoreType.DMA,
        pltpu.SemaphoreType.DMA,
        pltpu.SemaphoreType.DMA,
    ],
)


def kernel(x, edge_index, W_enc, b_enc, ln_w1, ln_b1, Wm1_1, bm1_1, Wm2_1,
           bm2_1, ln_w2, ln_b2, Wm1_2, bm1_2, Wm2_2, bm2_2, ln_w3, ln_b3,
           Wm1_3, bm1_3, Wm2_3, bm2_3, W_cls, b_cls):
    f32 = jnp.float32
    src = edge_index[0]
    dst = edge_index[1]
    pad = EPAD - E
    srcp = jnp.concatenate([src, jnp.zeros((pad,), jnp.int32)])
    dstp = jnp.concatenate(
        [dst, N + (jnp.arange(pad, dtype=jnp.int32) % (NPAD - N))])
    srcs = srcp.reshape(NS, CHUNKS, CHUNK)
    dsts = dstp.reshape(NS, CHUNKS, CHUNK)
    zeros = jnp.zeros((NPAD, D), f32)

    r = lambda v: v.reshape(1, -1).astype(f32)

    h, t, em, p = _enc_pre(x, W_enc, r(b_enc), r(ln_w1), r(ln_b1))
    den, num = _sc_agg(em, p, srcs, dsts, zeros)
    h, t, em, p = _post_pre(h, t, den, num, Wm1_1, r(bm1_1), Wm2_1, r(bm2_1),
                            r(ln_w2), r(ln_b2))
    den, num = _sc_agg(em, p, srcs, dsts, zeros)
    h, t, em, p = _post_pre(h, t, den, num, Wm1_2, r(bm1_2), Wm2_2, r(bm2_2),
                            r(ln_w3), r(ln_b3))
    den, num = _sc_agg(em, p, srcs, dsts, zeros)
    return _post_cls(h, t, den, num, Wm1_3, r(bm1_3), Wm2_3, r(bm2_3),
                     W_cls, r(b_cls))
